# baseline (device time: 17141 ns/iter reference)
import jax
import jax.numpy as jnp
from jax import lax
from jax.experimental import pallas as pl
from jax.experimental.pallas import tpu as pltpu

N_DEV = 4
EPS = 1e-5
GLOBAL_HW = 512 * 128
NCHUNK = 4


def kernel(x, Wp):
    xt = x.transpose(0, 1, 3, 2)
    b, h_per, c, w = xt.shape
    c_out = Wp.shape[1]
    hc = h_per // NCHUNK

    def body(xt_hbm, wp_hbm, out_ref, xv, xbv, wp_v, own_ref, stats_ref,
             in_sems, wp_sem, send_sems, recv_sems):
        my = lax.axis_index("i")

        def in_copy(i):
            return pltpu.make_async_copy(
                xt_hbm.at[:, pl.ds(i * hc, hc)],
                xv.at[:, pl.ds(i * hc, hc)],
                in_sems.at[i],
            )

        with jax.named_scope("issue_in"):
            for i in range(NCHUNK):
                in_copy(i).start()
            wp_cp = pltpu.make_async_copy(wp_hbm, wp_v, wp_sem.at[0])
            wp_cp.start()

        with jax.named_scope("barrier"):
            barrier_sem = pltpu.get_barrier_semaphore()
            for d in range(1, N_DEV):
                pl.semaphore_signal(
                    barrier_sem, inc=1,
                    device_id=((my + d) % N_DEV,),
                    device_id_type=pl.DeviceIdType.MESH,
                )
            pl.semaphore_wait(barrier_sem, N_DEV - 1)

        with jax.named_scope("stats"):
            s1 = jnp.zeros((b, c), jnp.float32)
            s2 = jnp.zeros((b, c), jnp.float32)
            for i in range(NCHUNK):
                in_copy(i).wait()
                xc = xv[:, i * hc:(i + 1) * hc]
                s1 = s1 + jnp.sum(xc, axis=(1, 3))
                s2 = s2 + jnp.sum(xc * xc, axis=(1, 3))
            own_ref[0, :, :] = s1
            own_ref[1, :, :] = s2

        with jax.named_scope("rdma_start"):
            for d in range(1, N_DEV):
                rdma = pltpu.make_async_remote_copy(
                    src_ref=own_ref,
                    dst_ref=stats_ref.at[d - 1],
                    send_sem=send_sems.at[d - 1],
                    recv_sem=recv_sems.at[d - 1],
                    device_id=((my + d) % N_DEV,),
                    device_id_type=pl.DeviceIdType.MESH,
                )
                rdma.start()

        with jax.named_scope("cast"):
            xbv[...] = xv[...].astype(jnp.bfloat16)

        with jax.named_scope("rdma_wait"):
            for k in range(N_DEV - 1):
                recv = pltpu.make_async_remote_copy(
                    src_ref=own_ref,
                    dst_ref=stats_ref.at[k],
                    send_sem=send_sems.at[k],
                    recv_sem=recv_sems.at[k],
                    device_id=(my,),
                    device_id_type=pl.DeviceIdType.MESH,
                )
                recv.wait_recv()

        with jax.named_scope("finalize_stats"):
            tot1 = s1 + stats_ref[0, 0] + stats_ref[1, 0] + stats_ref[2, 0]
            tot2 = s2 + stats_ref[0, 1] + stats_ref[1, 1] + stats_ref[2, 1]
            mean = tot1 / GLOBAL_HW
            var = tot2 / GLOBAL_HW - mean * mean
            inv = lax.rsqrt(var + EPS)
            mean4 = mean.astype(jnp.bfloat16)[:, None, :, None]
            inv4 = inv.astype(jnp.bfloat16)[:, None, :, None]

            wp_cp.wait()
            wb = wp_v[...].astype(jnp.bfloat16)

        with jax.named_scope("p2"):
            xs = xbv[...]
            hn = (xs - mean4) * inv4
            a = (hn * jax.nn.sigmoid(hn)).astype(jnp.bfloat16)
            at = jnp.swapaxes(a.reshape(b * h_per, c, w), 1, 2)
            a2 = at.reshape(b * h_per * w, c)
            y = jnp.dot(a2, wb, preferred_element_type=jnp.float32)
            out_ref[...] = y.reshape(b, h_per, w, c_out).astype(jnp.bfloat16)

        for k in range(N_DEV - 1):
            snd = pltpu.make_async_remote_copy(
                src_ref=own_ref,
                dst_ref=stats_ref.at[k],
                send_sem=send_sems.at[k],
                recv_sem=recv_sems.at[k],
                device_id=(my,),
                device_id_type=pl.DeviceIdType.MESH,
            )
            snd.wait_send()

    return pl.pallas_call(
        body,
        out_shape=jax.ShapeDtypeStruct((b, h_per, w, c_out), jnp.bfloat16),
        in_specs=[
            pl.BlockSpec(memory_space=pltpu.HBM),
            pl.BlockSpec(memory_space=pltpu.HBM),
        ],
        out_specs=pl.BlockSpec(memory_space=pltpu.VMEM),
        scratch_shapes=[
            pltpu.VMEM((b, h_per, c, w), jnp.float32),
            pltpu.VMEM((b, h_per, c, w), jnp.bfloat16),
            pltpu.VMEM((c, c_out), jnp.float32),
            pltpu.VMEM((2, b, c), jnp.float32),
            pltpu.VMEM((N_DEV - 1, 2, b, c), jnp.float32),
            pltpu.SemaphoreType.DMA((NCHUNK,)),
            pltpu.SemaphoreType.DMA((1,)),
            pltpu.SemaphoreType.DMA((N_DEV - 1,)),
            pltpu.SemaphoreType.DMA((N_DEV - 1,)),
        ],
        compiler_params=pltpu.CompilerParams(collective_id=0),
    )(
        pltpu.with_memory_space_constraint(xt, pltpu.HBM),
        pltpu.with_memory_space_constraint(Wp, pltpu.HBM),
    )


# device time: 15257 ns/iter; 1.1235x vs baseline; 1.1235x over previous
import jax
import jax.numpy as jnp
from jax import lax
from jax.experimental import pallas as pl
from jax.experimental.pallas import tpu as pltpu

N_DEV = 4
EPS = 1e-5
GLOBAL_HW = 512 * 128
NCHUNK = 4


def kernel(x, Wp):
    xt = x.transpose(0, 1, 3, 2)
    b, h_per, c, w = xt.shape
    c_out = Wp.shape[1]
    hc = h_per // NCHUNK

    def body(xt_hbm, wp_hbm, out_ref, xv, wp_v, own_ref, stats_ref,
             in_sems, wp_sem, send_sems, recv_sems):
        my = lax.axis_index("i")

        def in_copy(i):
            return pltpu.make_async_copy(
                xt_hbm.at[:, pl.ds(i * hc, hc)],
                xv.at[:, pl.ds(i * hc, hc)],
                in_sems.at[i],
            )

        with jax.named_scope("issue_in"):
            for i in range(NCHUNK):
                in_copy(i).start()
            wp_cp = pltpu.make_async_copy(wp_hbm, wp_v, wp_sem.at[0])
            wp_cp.start()

        with jax.named_scope("barrier"):
            barrier_sem = pltpu.get_barrier_semaphore()
            for d in range(1, N_DEV):
                pl.semaphore_signal(
                    barrier_sem, inc=1,
                    device_id=((my + d) % N_DEV,),
                    device_id_type=pl.DeviceIdType.MESH,
                )
            pl.semaphore_wait(barrier_sem, N_DEV - 1)

        with jax.named_scope("stats"):
            s1 = jnp.zeros((b, c), jnp.float32)
            s2 = jnp.zeros((b, c), jnp.float32)
            for i in range(NCHUNK):
                in_copy(i).wait()
                xc = xv[:, i * hc:(i + 1) * hc]
                s1 = s1 + jnp.sum(xc, axis=(1, 3))
                s2 = s2 + jnp.sum(xc * xc, axis=(1, 3))
            own_ref[0, :, :] = s1
            own_ref[1, :, :] = s2

        with jax.named_scope("rdma_start"):
            for d in range(1, N_DEV):
                rdma = pltpu.make_async_remote_copy(
                    src_ref=own_ref,
                    dst_ref=stats_ref.at[d - 1],
                    send_sem=send_sems.at[d - 1],
                    recv_sem=recv_sems.at[d - 1],
                    device_id=((my + d) % N_DEV,),
                    device_id_type=pl.DeviceIdType.MESH,
                )
                rdma.start()

        with jax.named_scope("rdma_wait"):
            for k in range(N_DEV - 1):
                recv = pltpu.make_async_remote_copy(
                    src_ref=own_ref,
                    dst_ref=stats_ref.at[k],
                    send_sem=send_sems.at[k],
                    recv_sem=recv_sems.at[k],
                    device_id=(my,),
                    device_id_type=pl.DeviceIdType.MESH,
                )
                recv.wait_recv()

        with jax.named_scope("finalize_stats"):
            tot1 = s1 + stats_ref[0, 0] + stats_ref[1, 0] + stats_ref[2, 0]
            tot2 = s2 + stats_ref[0, 1] + stats_ref[1, 1] + stats_ref[2, 1]
            mean = tot1 / GLOBAL_HW
            var = tot2 / GLOBAL_HW - mean * mean
            inv = lax.rsqrt(var + EPS)
            mean4 = mean[:, None, :, None]
            inv4 = inv[:, None, :, None]

            wp_cp.wait()
            wb = wp_v[...].astype(jnp.bfloat16)

        with jax.named_scope("p2"):
            xs = xv[...]
            hn = (xs - mean4) * inv4
            a = (hn * jax.nn.sigmoid(hn)).astype(jnp.bfloat16)
            at = jnp.swapaxes(a.reshape(b * h_per, c, w), 1, 2)
            a2 = at.reshape(b * h_per * w, c)
            y = jnp.dot(a2, wb, preferred_element_type=jnp.float32)
            out_ref[...] = y.reshape(b, h_per, w, c_out).astype(jnp.bfloat16)

        for k in range(N_DEV - 1):
            snd = pltpu.make_async_remote_copy(
                src_ref=own_ref,
                dst_ref=stats_ref.at[k],
                send_sem=send_sems.at[k],
                recv_sem=recv_sems.at[k],
                device_id=(my,),
                device_id_type=pl.DeviceIdType.MESH,
            )
            snd.wait_send()

    return pl.pallas_call(
        body,
        out_shape=jax.ShapeDtypeStruct((b, h_per, w, c_out), jnp.bfloat16),
        in_specs=[
            pl.BlockSpec(memory_space=pltpu.HBM),
            pl.BlockSpec(memory_space=pltpu.HBM),
        ],
        out_specs=pl.BlockSpec(memory_space=pltpu.VMEM),
        scratch_shapes=[
            pltpu.VMEM((b, h_per, c, w), jnp.float32),
            pltpu.VMEM((c, c_out), jnp.float32),
            pltpu.VMEM((2, b, c), jnp.float32),
            pltpu.VMEM((N_DEV - 1, 2, b, c), jnp.float32),
            pltpu.SemaphoreType.DMA((NCHUNK,)),
            pltpu.SemaphoreType.DMA((1,)),
            pltpu.SemaphoreType.DMA((N_DEV - 1,)),
            pltpu.SemaphoreType.DMA((N_DEV - 1,)),
        ],
        compiler_params=pltpu.CompilerParams(collective_id=0),
    )(
        pltpu.with_memory_space_constraint(xt, pltpu.HBM),
        pltpu.with_memory_space_constraint(Wp, pltpu.HBM),
    )


# device time: 15178 ns/iter; 1.1293x vs baseline; 1.0052x over previous
import jax
import jax.numpy as jnp
from jax import lax
from jax.experimental import pallas as pl
from jax.experimental.pallas import tpu as pltpu

N_DEV = 4
EPS = 1e-5
GLOBAL_HW = 512 * 128
NCHUNK = 8


def kernel(x, Wp):
    xt = x.transpose(0, 1, 3, 2)
    b, h_per, c, w = xt.shape
    c_out = Wp.shape[1]
    hc = h_per // NCHUNK

    def body(xt_hbm, wp_hbm, out_ref, xv, wp_v, own_ref, stats_ref,
             in_sems, wp_sem, send_sems, recv_sems):
        my = lax.axis_index("i")

        def in_copy(i):
            return pltpu.make_async_copy(
                xt_hbm.at[:, pl.ds(i * hc, hc)],
                xv.at[:, pl.ds(i * hc, hc)],
                in_sems.at[i],
            )

        with jax.named_scope("issue_in"):
            for i in range(NCHUNK):
                in_copy(i).start()
            wp_cp = pltpu.make_async_copy(wp_hbm, wp_v, wp_sem.at[0])
            wp_cp.start()

        with jax.named_scope("barrier"):
            barrier_sem = pltpu.get_barrier_semaphore()
            for d in range(1, N_DEV):
                pl.semaphore_signal(
                    barrier_sem, inc=1,
                    device_id=((my + d) % N_DEV,),
                    device_id_type=pl.DeviceIdType.MESH,
                )
            pl.semaphore_wait(barrier_sem, N_DEV - 1)

        with jax.named_scope("stats"):
            s1 = jnp.zeros((b, c), jnp.float32)
            s2 = jnp.zeros((b, c), jnp.float32)
            for i in range(NCHUNK):
                in_copy(i).wait()
                xc = xv[:, i * hc:(i + 1) * hc]
                s1 = s1 + jnp.sum(xc, axis=(1, 3))
                s2 = s2 + jnp.sum(xc * xc, axis=(1, 3))
            own_ref[0, :, :] = s1
            own_ref[1, :, :] = s2

        with jax.named_scope("rdma_start"):
            for d in range(1, N_DEV):
                rdma = pltpu.make_async_remote_copy(
                    src_ref=own_ref,
                    dst_ref=stats_ref.at[d - 1],
                    send_sem=send_sems.at[d - 1],
                    recv_sem=recv_sems.at[d - 1],
                    device_id=((my + d) % N_DEV,),
                    device_id_type=pl.DeviceIdType.MESH,
                )
                rdma.start()

        with jax.named_scope("rdma_wait"):
            for k in range(N_DEV - 1):
                recv = pltpu.make_async_remote_copy(
                    src_ref=own_ref,
                    dst_ref=stats_ref.at[k],
                    send_sem=send_sems.at[k],
                    recv_sem=recv_sems.at[k],
                    device_id=(my,),
                    device_id_type=pl.DeviceIdType.MESH,
                )
                recv.wait_recv()

        with jax.named_scope("finalize_stats"):
            tot1 = s1 + stats_ref[0, 0] + stats_ref[1, 0] + stats_ref[2, 0]
            tot2 = s2 + stats_ref[0, 1] + stats_ref[1, 1] + stats_ref[2, 1]
            mean = tot1 / GLOBAL_HW
            var = tot2 / GLOBAL_HW - mean * mean
            inv = lax.rsqrt(var + EPS)
            mean4 = mean[:, None, :, None]
            inv4 = inv[:, None, :, None]

            wp_cp.wait()
            wb = wp_v[...].astype(jnp.bfloat16)

        P2C = 4
        h2 = h_per // P2C
        for i in range(P2C):
            with jax.named_scope(f"p2#chunk={i}"):
                xs = xv[:, i * h2:(i + 1) * h2]
                hn = (xs - mean4) * inv4
                a = (hn * jax.nn.sigmoid(hn)).astype(jnp.bfloat16)
                at = jnp.swapaxes(a.reshape(b * h2, c, w), 1, 2)
                a2 = at.reshape(b * h2 * w, c)
                y = jnp.dot(a2, wb, preferred_element_type=jnp.float32)
                out_ref[:, i * h2:(i + 1) * h2] = (
                    y.reshape(b, h2, w, c_out).astype(jnp.bfloat16))

        for k in range(N_DEV - 1):
            snd = pltpu.make_async_remote_copy(
                src_ref=own_ref,
                dst_ref=stats_ref.at[k],
                send_sem=send_sems.at[k],
                recv_sem=recv_sems.at[k],
                device_id=(my,),
                device_id_type=pl.DeviceIdType.MESH,
            )
            snd.wait_send()

    return pl.pallas_call(
        body,
        out_shape=jax.ShapeDtypeStruct((b, h_per, w, c_out), jnp.bfloat16),
        in_specs=[
            pl.BlockSpec(memory_space=pltpu.HBM),
            pl.BlockSpec(memory_space=pltpu.HBM),
        ],
        out_specs=pl.BlockSpec(memory_space=pltpu.VMEM),
        scratch_shapes=[
            pltpu.VMEM((b, h_per, c, w), jnp.float32),
            pltpu.VMEM((c, c_out), jnp.float32),
            pltpu.VMEM((2, b, c), jnp.float32),
            pltpu.VMEM((N_DEV - 1, 2, b, c), jnp.float32),
            pltpu.SemaphoreType.DMA((NCHUNK,)),
            pltpu.SemaphoreType.DMA((1,)),
            pltpu.SemaphoreType.DMA((N_DEV - 1,)),
            pltpu.SemaphoreType.DMA((N_DEV - 1,)),
        ],
        compiler_params=pltpu.CompilerParams(collective_id=0),
    )(
        pltpu.with_memory_space_constraint(xt, pltpu.HBM),
        pltpu.with_memory_space_constraint(Wp, pltpu.HBM),
    )
